# baseline (device time: 164195 ns/iter reference)
import jax
import jax.numpy as jnp
from jax import lax
from jax.experimental import pallas as pl
from jax.experimental.pallas import tpu as pltpu

NZ = 4


def kernel(Q, K, V):
    b, s, h, d = Q.shape
    scale = d ** -0.5

    def body(q_ref, k_ref, v_ref, o_ref, kbuf, vbuf,
             ksend, krecv, vsend, vrecv):
        my_x = lax.axis_index("x")
        my_y = lax.axis_index("y")
        my_z = lax.axis_index("z")
        right = (my_z + 1) % NZ
        left = (my_z - 1) % NZ

        barrier = pltpu.get_barrier_semaphore()
        pl.semaphore_signal(barrier, inc=1, device_id=(my_x, my_y, left),
                            device_id_type=pl.DeviceIdType.MESH)
        pl.semaphore_signal(barrier, inc=1, device_id=(my_x, my_y, right),
                            device_id_type=pl.DeviceIdType.MESH)
        pl.semaphore_wait(barrier, 2)

        kbuf[my_z] = k_ref[...]
        vbuf[my_z] = v_ref[...]

        for hop in range(NZ - 1):
            ok = (my_z - hop) % NZ
            ov = (my_z + hop) % NZ
            rk = pltpu.make_async_remote_copy(
                src_ref=kbuf.at[ok], dst_ref=kbuf.at[ok],
                send_sem=ksend.at[hop], recv_sem=krecv.at[hop],
                device_id=(my_x, my_y, right),
                device_id_type=pl.DeviceIdType.MESH)
            rv = pltpu.make_async_remote_copy(
                src_ref=vbuf.at[ov], dst_ref=vbuf.at[ov],
                send_sem=vsend.at[hop], recv_sem=vrecv.at[hop],
                device_id=(my_x, my_y, left),
                device_id_type=pl.DeviceIdType.MESH)
            rk.start()
            rv.start()
            rk.wait()
            rv.wait()

        for bi in range(b):
            for hi in range(h):
                q = q_ref[bi, :, hi, :]
                kf = jnp.concatenate(
                    [kbuf[o, bi, :, hi, :] for o in range(NZ)], axis=0)
                vf = jnp.concatenate(
                    [vbuf[o, bi, :, hi, :] for o in range(NZ)], axis=0)
                s_mat = lax.dot_general(
                    q, kf, (((1,), (1,)), ((), ())),
                    preferred_element_type=jnp.float32) * scale
                m = jnp.max(s_mat, axis=1, keepdims=True)
                p = jnp.exp(s_mat - m)
                p = p / jnp.sum(p, axis=1, keepdims=True)
                o_ref[bi, :, hi, :] = lax.dot_general(
                    p, vf, (((1,), (0,)), ((), ())),
                    preferred_element_type=jnp.float32)

    return pl.pallas_call(
        body,
        out_shape=jax.ShapeDtypeStruct((b, s, h, d), jnp.float32),
        in_specs=[pl.BlockSpec(memory_space=pltpu.VMEM)] * 3,
        out_specs=pl.BlockSpec(memory_space=pltpu.VMEM),
        scratch_shapes=[
            pltpu.VMEM((NZ, b, s, h, d), jnp.float32),
            pltpu.VMEM((NZ, b, s, h, d), jnp.float32),
            pltpu.SemaphoreType.DMA((NZ - 1,)),
            pltpu.SemaphoreType.DMA((NZ - 1,)),
            pltpu.SemaphoreType.DMA((NZ - 1,)),
            pltpu.SemaphoreType.DMA((NZ - 1,)),
        ],
        compiler_params=pltpu.CompilerParams(collective_id=0),
    )(Q, K, V)


# device time: 155280 ns/iter; 1.0574x vs baseline; 1.0574x over previous
import jax
import jax.numpy as jnp
from jax import lax
from jax.experimental import pallas as pl
from jax.experimental.pallas import tpu as pltpu

NZ = 4


def kernel(Q, K, V):
    b, s, h, d = Q.shape
    scale = d ** -0.5

    def body(q_ref, k_ref, v_ref, o_ref, kbuf, vbuf,
             ksend, krecv, vsend, vrecv):
        my_x = lax.axis_index("x")
        my_y = lax.axis_index("y")
        my_z = lax.axis_index("z")
        right = (my_z + 1) % NZ
        left = (my_z - 1) % NZ

        barrier = pltpu.get_barrier_semaphore()
        pl.semaphore_signal(barrier, inc=1, device_id=(my_x, my_y, left),
                            device_id_type=pl.DeviceIdType.MESH)
        pl.semaphore_signal(barrier, inc=1, device_id=(my_x, my_y, right),
                            device_id_type=pl.DeviceIdType.MESH)
        pl.semaphore_wait(barrier, 2)

        kbuf[my_z] = k_ref[...]
        vbuf[my_z] = v_ref[...]

        for hop in range(NZ - 1):
            ok = (my_z - hop) % NZ
            ov = (my_z + hop) % NZ
            rk = pltpu.make_async_remote_copy(
                src_ref=kbuf.at[ok], dst_ref=kbuf.at[ok],
                send_sem=ksend.at[hop], recv_sem=krecv.at[hop],
                device_id=(my_x, my_y, right),
                device_id_type=pl.DeviceIdType.MESH)
            rv = pltpu.make_async_remote_copy(
                src_ref=vbuf.at[ov], dst_ref=vbuf.at[ov],
                send_sem=vsend.at[hop], recv_sem=vrecv.at[hop],
                device_id=(my_x, my_y, left),
                device_id_type=pl.DeviceIdType.MESH)
            rk.start()
            rv.start()
            rk.wait()
            rv.wait()

        for bi in range(b):
            for hi in range(h):
                q = q_ref[bi, hi]
                kf = jnp.concatenate(
                    [kbuf[o, bi, hi] for o in range(NZ)], axis=0)
                vf = jnp.concatenate(
                    [vbuf[o, bi, hi] for o in range(NZ)], axis=0)
                s_mat = lax.dot_general(
                    q, kf, (((1,), (1,)), ((), ())),
                    preferred_element_type=jnp.float32) * scale
                m = jnp.max(s_mat, axis=1, keepdims=True)
                p = jnp.exp(s_mat - m)
                p = p / jnp.sum(p, axis=1, keepdims=True)
                o_ref[bi, hi] = lax.dot_general(
                    p, vf, (((1,), (0,)), ((), ())),
                    preferred_element_type=jnp.float32)

    Qt = jnp.transpose(Q, (0, 2, 1, 3))
    Kt = jnp.transpose(K, (0, 2, 1, 3))
    Vt = jnp.transpose(V, (0, 2, 1, 3))

    out = pl.pallas_call(
        body,
        out_shape=jax.ShapeDtypeStruct((b, h, s, d), jnp.float32),
        in_specs=[pl.BlockSpec(memory_space=pltpu.VMEM)] * 3,
        out_specs=pl.BlockSpec(memory_space=pltpu.VMEM),
        scratch_shapes=[
            pltpu.VMEM((NZ, b, h, s, d), jnp.float32),
            pltpu.VMEM((NZ, b, h, s, d), jnp.float32),
            pltpu.SemaphoreType.DMA((NZ - 1,)),
            pltpu.SemaphoreType.DMA((NZ - 1,)),
            pltpu.SemaphoreType.DMA((NZ - 1,)),
            pltpu.SemaphoreType.DMA((NZ - 1,)),
        ],
        compiler_params=pltpu.CompilerParams(collective_id=0),
    )(Qt, Kt, Vt)
    return jnp.transpose(out, (0, 2, 1, 3))


# device time: 148676 ns/iter; 1.1044x vs baseline; 1.0444x over previous
import os

import jax
import jax.numpy as jnp
from jax import lax
from jax.experimental import pallas as pl
from jax.experimental.pallas import tpu as pltpu

NZ = 4
_MODE = os.environ.get("KMODE", "full")


def kernel(Q, K, V):
    b, s, h, d = Q.shape
    scale = d ** -0.5

    def body(q_ref, k_ref, v_ref, o_ref, kbuf, vbuf,
             ksend, krecv, vsend, vrecv):
        my_x = lax.axis_index("x")
        my_y = lax.axis_index("y")
        my_z = lax.axis_index("z")
        right = (my_z + 1) % NZ
        left = (my_z - 1) % NZ

        barrier = pltpu.get_barrier_semaphore()
        pl.semaphore_signal(barrier, inc=1, device_id=(my_x, my_y, left),
                            device_id_type=pl.DeviceIdType.MESH)
        pl.semaphore_signal(barrier, inc=1, device_id=(my_x, my_y, right),
                            device_id_type=pl.DeviceIdType.MESH)
        pl.semaphore_wait(barrier, 2)

        kbuf[my_z] = k_ref[...]
        vbuf[my_z] = v_ref[...]

        for hop in range(NZ - 1) if _MODE != "compute_only" else []:
            ok = (my_z - hop) % NZ
            ov = (my_z + hop) % NZ
            rk = pltpu.make_async_remote_copy(
                src_ref=kbuf.at[ok], dst_ref=kbuf.at[ok],
                send_sem=ksend.at[hop], recv_sem=krecv.at[hop],
                device_id=(my_x, my_y, right),
                device_id_type=pl.DeviceIdType.MESH)
            rv = pltpu.make_async_remote_copy(
                src_ref=vbuf.at[ov], dst_ref=vbuf.at[ov],
                send_sem=vsend.at[hop], recv_sem=vrecv.at[hop],
                device_id=(my_x, my_y, left),
                device_id_type=pl.DeviceIdType.MESH)
            rk.start()
            rv.start()
            rk.wait()
            rv.wait()

        if _MODE == "comm_only":
            o_ref[...] = q_ref[...]
            return
        for bi in range(b):
            for hi in range(h):
                q = q_ref[bi, hi]
                kf = jnp.concatenate(
                    [kbuf[o, bi, hi] for o in range(NZ)], axis=0)
                vf = jnp.concatenate(
                    [vbuf[o, bi, hi] for o in range(NZ)], axis=0)
                s_mat = lax.dot_general(
                    q, kf, (((1,), (1,)), ((), ())),
                    preferred_element_type=jnp.float32) * scale
                m = jnp.max(s_mat, axis=1, keepdims=True)
                p = jnp.exp(s_mat - m)
                p = p / jnp.sum(p, axis=1, keepdims=True)
                o_ref[bi, hi] = lax.dot_general(
                    p, vf, (((1,), (0,)), ((), ())),
                    preferred_element_type=jnp.float32)

    Qt = jnp.transpose(Q, (0, 2, 1, 3))
    Kt = jnp.transpose(K, (0, 2, 1, 3))
    Vt = jnp.transpose(V, (0, 2, 1, 3))

    out = pl.pallas_call(
        body,
        out_shape=jax.ShapeDtypeStruct((b, h, s, d), jnp.float32),
        in_specs=[pl.BlockSpec(memory_space=pltpu.VMEM)] * 3,
        out_specs=pl.BlockSpec(memory_space=pltpu.VMEM),
        scratch_shapes=[
            pltpu.VMEM((NZ, b, h, s, d), jnp.float32),
            pltpu.VMEM((NZ, b, h, s, d), jnp.float32),
            pltpu.SemaphoreType.DMA((NZ - 1,)),
            pltpu.SemaphoreType.DMA((NZ - 1,)),
            pltpu.SemaphoreType.DMA((NZ - 1,)),
            pltpu.SemaphoreType.DMA((NZ - 1,)),
        ],
        compiler_params=pltpu.CompilerParams(collective_id=0),
    )(Qt, Kt, Vt)
    return jnp.transpose(out, (0, 2, 1, 3))


# device time: 88814 ns/iter; 1.8488x vs baseline; 1.6740x over previous
import os

import jax
import jax.numpy as jnp
from jax import lax
from jax.experimental import pallas as pl
from jax.experimental.pallas import tpu as pltpu

NZ = 4
_MODE = os.environ.get("KMODE", "full")


def kernel(Q, K, V):
    b, s, h, d = Q.shape
    scale = d ** -0.5

    def body(q_ref, k_ref, v_ref, o_ref, kbuf, vbuf,
             ksend, krecv, vsend, vrecv):
        my_x = lax.axis_index("x")
        my_y = lax.axis_index("y")
        my_z = lax.axis_index("z")
        right = (my_z + 1) % NZ
        left = (my_z - 1) % NZ

        barrier = pltpu.get_barrier_semaphore()
        pl.semaphore_signal(barrier, inc=1, device_id=(my_x, my_y, left),
                            device_id_type=pl.DeviceIdType.MESH)
        pl.semaphore_signal(barrier, inc=1, device_id=(my_x, my_y, right),
                            device_id_type=pl.DeviceIdType.MESH)
        pl.semaphore_wait(barrier, 2)

        kbuf[my_z] = k_ref[...]
        vbuf[my_z] = v_ref[...]

        for hop in range(NZ - 1) if _MODE != "compute_only" else []:
            ok = (my_z - hop) % NZ
            ov = (my_z + hop) % NZ
            rk = pltpu.make_async_remote_copy(
                src_ref=kbuf.at[ok], dst_ref=kbuf.at[ok],
                send_sem=ksend.at[hop], recv_sem=krecv.at[hop],
                device_id=(my_x, my_y, right),
                device_id_type=pl.DeviceIdType.MESH)
            rv = pltpu.make_async_remote_copy(
                src_ref=vbuf.at[ov], dst_ref=vbuf.at[ov],
                send_sem=vsend.at[hop], recv_sem=vrecv.at[hop],
                device_id=(my_x, my_y, left),
                device_id_type=pl.DeviceIdType.MESH)
            rk.start()
            if _MODE != "k_only":
                rv.start()
            rk.wait()
            if _MODE != "k_only":
                rv.wait()

        if _MODE == "comm_only":
            o_ref[...] = q_ref[...]
            return
        for bi in range(b):
            for hi in range(h):
                q = q_ref[bi, hi]
                kf = jnp.concatenate(
                    [kbuf[o, bi, hi] for o in range(NZ)], axis=0)
                vf = jnp.concatenate(
                    [vbuf[o, bi, hi] for o in range(NZ)], axis=0)
                s_mat = lax.dot_general(
                    q, kf, (((1,), (1,)), ((), ())),
                    preferred_element_type=jnp.float32) * scale
                m = jnp.max(s_mat, axis=1, keepdims=True)
                p = jnp.exp(s_mat - m)
                p = p / jnp.sum(p, axis=1, keepdims=True)
                o_ref[bi, hi] = lax.dot_general(
                    p, vf, (((1,), (0,)), ((), ())),
                    preferred_element_type=jnp.float32)

    Qt = jnp.transpose(Q, (0, 2, 1, 3))
    Kt = jnp.transpose(K, (0, 2, 1, 3))
    Vt = jnp.transpose(V, (0, 2, 1, 3))

    out = pl.pallas_call(
        body,
        out_shape=jax.ShapeDtypeStruct((b, h, s, d), jnp.float32),
        in_specs=[pl.BlockSpec(memory_space=pltpu.VMEM)] * 3,
        out_specs=pl.BlockSpec(memory_space=pltpu.VMEM),
        scratch_shapes=[
            pltpu.VMEM((NZ, b, h, s, d), jnp.float32),
            pltpu.VMEM((NZ, b, h, s, d), jnp.float32),
            pltpu.SemaphoreType.DMA((NZ - 1,)),
            pltpu.SemaphoreType.DMA((NZ - 1,)),
            pltpu.SemaphoreType.DMA((NZ - 1,)),
            pltpu.SemaphoreType.DMA((NZ - 1,)),
        ],
        compiler_params=pltpu.CompilerParams(collective_id=0),
    )(Qt, Kt, Vt)
    return jnp.transpose(out, (0, 2, 1, 3))


# device time: 86530 ns/iter; 1.8975x vs baseline; 1.0264x over previous
import os

import jax
import jax.numpy as jnp
from jax import lax
from jax.experimental import pallas as pl
from jax.experimental.pallas import tpu as pltpu

NZ = 4
_MODE = os.environ.get("KMODE", "full")
_MESH = pl.DeviceIdType.MESH


def kernel(Q, K, V):
    b, s, h, d = Q.shape
    scale = d ** -0.5
    half = s // 2

    def body(q_ref, kv_ref, o_ref, kvbuf, upsend, dnsend, xsd, ysd,
             yfwd, xfwd, zrecv, xrecvd, yrecvd, yrecvf, xrecvf):
        my_x = lax.axis_index("x")
        my_y = lax.axis_index("y")
        my_z = lax.axis_index("z")
        has_up = my_z < NZ - 1
        has_dn = my_z > 0
        up_dev = (my_x, my_y, jnp.minimum(my_z + 1, NZ - 1))
        dn_dev = (my_x, my_y, jnp.maximum(my_z - 1, 0))
        x_dev = (1 - my_x, my_y, my_z)
        y_dev = (my_x, 1 - my_y, my_z)
        self_dev = (my_x, my_y, my_z)

        bar = pltpu.get_barrier_semaphore()

        @pl.when(has_dn)
        def _():
            pl.semaphore_signal(bar, inc=1, device_id=dn_dev,
                                device_id_type=_MESH)

        @pl.when(has_up)
        def _():
            pl.semaphore_signal(bar, inc=1, device_id=up_dev,
                                device_id_type=_MESH)

        pl.semaphore_signal(bar, inc=1, device_id=x_dev, device_id_type=_MESH)
        pl.semaphore_signal(bar, inc=1, device_id=y_dev, device_id_type=_MESH)
        pl.semaphore_wait(bar, 2)

        @pl.when(has_dn)
        def _():
            pl.semaphore_wait(bar, 1)

        @pl.when(has_up)
        def _():
            pl.semaphore_wait(bar, 1)

        kvbuf[my_z] = kv_ref[...]

        sends = []

        def launch(cond, src, dev, ssem, rsem):
            c = pltpu.make_async_remote_copy(
                src_ref=src, dst_ref=src, send_sem=ssem, recv_sem=rsem,
                device_id=dev, device_id_type=_MESH)

            @pl.when(cond)
            def _():
                c.start()

            sends.append((cond, c))

        def wait_recv(cond, dst, rsem):
            c = pltpu.make_async_remote_copy(
                src_ref=dst, dst_ref=dst, send_sem=rsem, recv_sem=rsem,
                device_id=self_dev, device_id_type=_MESH)

            @pl.when(cond)
            def _():
                c.wait_recv()

        def remote_origins():
            for delta in range(1, NZ):
                yield (my_z - delta >= 0, jnp.maximum(my_z - delta, 0))
                yield (my_z + delta <= NZ - 1,
                       jnp.minimum(my_z + delta, NZ - 1))

        if _MODE != "compute_only":
            for t in range(NZ - 1):
                uo = my_z - t
                uoc = jnp.maximum(uo, 0)
                do = my_z + t
                doc = jnp.minimum(do, NZ - 1)
                launch(has_up & (uo >= 0), kvbuf.at[uoc, my_x, my_y],
                       up_dev, upsend.at[t], zrecv.at[uoc])
                launch(has_dn & (do <= NZ - 1), kvbuf.at[doc, my_x, my_y],
                       dn_dev, dnsend.at[t], zrecv.at[doc])
                rb = my_z - 1 - t
                rbc = jnp.maximum(rb, 0)
                ra = my_z + 1 + t
                rac = jnp.minimum(ra, NZ - 1)
                for cond, oc in ((rb >= 0, rbc), (ra <= NZ - 1, rac)):
                    wait_recv(cond, kvbuf.at[oc, my_x, my_y], zrecv.at[oc])
                    launch(cond, kvbuf.at[oc, my_x, my_y],
                           x_dev, xsd.at[oc], xrecvd.at[oc])
                    launch(cond, kvbuf.at[oc, my_x, my_y],
                           y_dev, ysd.at[oc], yrecvd.at[oc])

            for cond, o in remote_origins():
                wait_recv(cond, kvbuf.at[o, 1 - my_x, my_y], xrecvd.at[o])
                launch(cond, kvbuf.at[o, 1 - my_x, my_y, :, pl.ds(0, half)],
                       y_dev, yfwd.at[o], yrecvf.at[o])
                wait_recv(cond, kvbuf.at[o, my_x, 1 - my_y], yrecvd.at[o])
                launch(cond,
                       kvbuf.at[o, my_x, 1 - my_y, :, pl.ds(half, half)],
                       x_dev, xfwd.at[o], xrecvf.at[o])

            for cond, o in remote_origins():
                wait_recv(cond,
                          kvbuf.at[o, 1 - my_x, 1 - my_y, :, pl.ds(0, half)],
                          yrecvf.at[o])
                wait_recv(cond,
                          kvbuf.at[o, 1 - my_x, 1 - my_y, :,
                                   pl.ds(half, half)],
                          xrecvf.at[o])

            for cond, c in sends:
                @pl.when(cond)
                def _(c=c):
                    c.wait_send()

        if _MODE == "comm_only":
            o_ref[...] = q_ref[...]
            return

        for bi in range(b):
            for hi in range(h):
                q = q_ref[bi, hi]
                kf = jnp.concatenate(
                    [kvbuf[o, bi, 0, hi] for o in range(NZ)], axis=0)
                vf = jnp.concatenate(
                    [kvbuf[o, bi, 1, hi] for o in range(NZ)], axis=0)
                s_mat = lax.dot_general(
                    q, kf, (((1,), (1,)), ((), ())),
                    preferred_element_type=jnp.float32) * scale
                m = jnp.max(s_mat, axis=1, keepdims=True)
                p = jnp.exp(s_mat - m)
                p = p / jnp.sum(p, axis=1, keepdims=True)
                o_ref[bi, hi] = lax.dot_general(
                    p, vf, (((1,), (0,)), ((), ())),
                    preferred_element_type=jnp.float32)

    Qt = jnp.transpose(Q, (0, 2, 1, 3))
    Kt = jnp.transpose(K, (0, 2, 1, 3))
    Vt = jnp.transpose(V, (0, 2, 1, 3))
    KV = jnp.stack((Kt, Vt), axis=1)

    out = pl.pallas_call(
        body,
        out_shape=jax.ShapeDtypeStruct((b, h, s, d), jnp.float32),
        in_specs=[pl.BlockSpec(memory_space=pltpu.VMEM)] * 2,
        out_specs=pl.BlockSpec(memory_space=pltpu.VMEM),
        scratch_shapes=[
            pltpu.VMEM((NZ, b, 2, h, s, d), jnp.float32),
            pltpu.SemaphoreType.DMA((NZ - 1,)),
            pltpu.SemaphoreType.DMA((NZ - 1,)),
            pltpu.SemaphoreType.DMA((NZ,)),
            pltpu.SemaphoreType.DMA((NZ,)),
            pltpu.SemaphoreType.DMA((NZ,)),
            pltpu.SemaphoreType.DMA((NZ,)),
            pltpu.SemaphoreType.DMA((NZ,)),
            pltpu.SemaphoreType.DMA((NZ,)),
            pltpu.SemaphoreType.DMA((NZ,)),
            pltpu.SemaphoreType.DMA((NZ,)),
            pltpu.SemaphoreType.DMA((NZ,)),
        ],
        compiler_params=pltpu.CompilerParams(collective_id=0),
    )(Qt, KV)
    return jnp.transpose(out, (0, 2, 1, 3))
